# Initial kernel scaffold; baseline (speedup 1.0000x reference)
import functools

import jax
import jax.numpy as jnp
from jax import lax
from jax.experimental import pallas as pl
from jax.experimental.pallas import tpu as pltpu
from jax.experimental.pallas import tpu_sc as plsc

L = 16
NWORK = 32
FPW = 4
CH = 3200
CH2 = 2000

_ALPHA = 1.0
_KORD = 3


def _wid():
    c = lax.axis_index("c")
    s = lax.axis_index("s")
    return s * 2 + c


def _rsqrt16(v):
    iv = lax.bitcast_convert_type(v, jnp.int32)
    y = lax.bitcast_convert_type(jnp.int32(0x5F3759DF) - (iv >> 1), jnp.float32)
    for _ in range(3):
        y = y * (1.5 - 0.5 * v * y * y)
    return y


def _degnorm_body(src_hbm, dst_hbm, ew_hbm, norm_hbm,
                  do_v, di_v, ro_v, ri_v, se, de, we, se2, de2, we2, nb2):
    E = src_hbm.shape[0]
    N = do_v.shape[0]
    wid = _wid()

    zeros = jnp.zeros((L,), jnp.float32)

    def z_body(i, _):
        do_v[pl.ds(i * L, L)] = zeros
        di_v[pl.ds(i * L, L)] = zeros
        return _
    lax.fori_loop(0, N // L, z_body, None)

    def ch_body(ci, _):
        off = ci * CH
        pltpu.sync_copy(src_hbm.at[pl.ds(off, CH)], se)
        pltpu.sync_copy(dst_hbm.at[pl.ds(off, CH)], de)
        pltpu.sync_copy(ew_hbm.at[pl.ds(off, CH)], we)

        def g_body(g, _):
            sl = pl.ds(g * L, L)
            w16 = we[sl]
            plsc.addupdate_scatter(do_v, [se[sl]], w16)
            plsc.addupdate_scatter(di_v, [de[sl]], w16)
            return _
        lax.fori_loop(0, CH // L, g_body, None)
        return _
    lax.fori_loop(0, E // CH, ch_body, None)

    def r_body(i, _):
        sl = pl.ds(i * L, L)
        ro_v[sl] = _rsqrt16(do_v[sl] + 1e-6)
        ri_v[sl] = _rsqrt16(di_v[sl] + 1e-6)
        return _
    lax.fori_loop(0, N // L, r_body, None)

    ew_per = E // NWORK

    def n_body(ci, _):
        off = wid * ew_per + ci * CH2
        pltpu.sync_copy(src_hbm.at[pl.ds(off, CH2)], se2)
        pltpu.sync_copy(dst_hbm.at[pl.ds(off, CH2)], de2)
        pltpu.sync_copy(ew_hbm.at[pl.ds(off, CH2)], we2)

        def g_body(g, _):
            sl = pl.ds(g * L, L)
            n16 = (we2[sl]
                   * plsc.load_gather(ro_v, [se2[sl]])
                   * plsc.load_gather(ri_v, [de2[sl]]))
            nb2[sl] = n16
            return _
        lax.fori_loop(0, CH2 // L, g_body, None)
        pltpu.sync_copy(nb2, norm_hbm.at[pl.ds(off, CH2)])
        return _
    lax.fori_loop(0, ew_per // CH2, n_body, None)


@functools.lru_cache(maxsize=None)
def _make_degnorm(E, N):
    mesh = plsc.VectorSubcoreMesh(core_axis_name="c", subcore_axis_name="s")
    f32, i32 = jnp.float32, jnp.int32
    return pl.kernel(
        _degnorm_body,
        out_type=jax.ShapeDtypeStruct((E,), f32),
        mesh=mesh,
        scratch_types=[
            pltpu.VMEM((N,), f32), pltpu.VMEM((N,), f32),
            pltpu.VMEM((N,), f32), pltpu.VMEM((N,), f32),
            pltpu.VMEM((CH,), i32), pltpu.VMEM((CH,), i32), pltpu.VMEM((CH,), f32),
            pltpu.VMEM((CH2,), i32), pltpu.VMEM((CH2,), i32), pltpu.VMEM((CH2,), f32),
            pltpu.VMEM((CH2,), f32),
        ],
    )


def _cascade_body(relu, h0_hbm, src_hbm, dst_hbm, norm_hbm, th_hbm, out_hbm,
                  t0, t1, t2, t3, u0, u1, u2, u3, c0, c1, c2, c3,
                  se, de, ne, th):
    E = src_hbm.shape[0]
    N = t0.shape[0]
    wid = _wid()
    ts = (t0, t1, t2, t3)
    us = (u0, u1, u2, u3)
    cs = (c0, c1, c2, c3)

    for j in range(FPW):
        pltpu.sync_copy(h0_hbm.at[wid * FPW + j], ts[j])
    pltpu.sync_copy(th_hbm, th)

    zeros = jnp.zeros((L,), jnp.float32)

    def z_body(i, _):
        for j in range(FPW):
            us[j][pl.ds(i * L, L)] = zeros
        return _
    lax.fori_loop(0, N // L, z_body, None)

    def prop(tables, accs, coef):
        def ch_body(ci, _):
            off = ci * CH
            pltpu.sync_copy(src_hbm.at[pl.ds(off, CH)], se)
            pltpu.sync_copy(dst_hbm.at[pl.ds(off, CH)], de)
            pltpu.sync_copy(norm_hbm.at[pl.ds(off, CH)], ne)

            def g_body(g, _):
                sl = pl.ds(g * L, L)
                s16 = se[sl]
                d16 = de[sl]
                n16 = ne[sl] * coef
                for j in range(FPW):
                    v = plsc.load_gather(tables[j], [s16]) * n16
                    plsc.addupdate_scatter(accs[j], [d16], v)
                return _
            lax.fori_loop(0, CH // L, g_body, None)
            return _
        lax.fori_loop(0, E // CH, ch_body, None)

    def ew_pass(fn):
        def body(i, _):
            fn(pl.ds(i * L, L))
            return _
        lax.fori_loop(0, N // L, body, None)

    th_v = [th[pl.ds(k * L, L)] for k in range(_KORD + 1)]

    prop(ts, us, 2.0 * _ALPHA)

    def comb_init(sl):
        for j in range(FPW):
            cs[j][sl] = th_v[0] * ts[j][sl] + th_v[1] * us[j][sl]
    ew_pass(comb_init)

    for k in range(2, _KORD + 1):
        a_k = 2.0 * (k - 1.0 + _ALPHA) / k
        b_k = (k - 2.0 + 2.0 * _ALPHA) / k
        tables, accs = (us, ts) if k % 2 == 0 else (ts, us)

        def pre(sl, accs=accs, b_k=b_k):
            for j in range(FPW):
                accs[j][sl] = accs[j][sl] * (-b_k)
        ew_pass(pre)
        prop(tables, accs, a_k)

        if k == _KORD and relu:
            def comb_last(sl, accs=accs, k=k):
                for j in range(FPW):
                    cs[j][sl] = jnp.maximum(
                        cs[j][sl] + th_v[k] * accs[j][sl], 0.0)
            ew_pass(comb_last)
        else:
            def comb_k(sl, accs=accs, k=k):
                for j in range(FPW):
                    cs[j][sl] = cs[j][sl] + th_v[k] * accs[j][sl]
            ew_pass(comb_k)

    for j in range(FPW):
        pltpu.sync_copy(cs[j], out_hbm.at[wid * FPW + j])


@functools.lru_cache(maxsize=None)
def _make_cascade(E, N, H, relu):
    mesh = plsc.VectorSubcoreMesh(core_axis_name="c", subcore_axis_name="s")
    f32, i32 = jnp.float32, jnp.int32
    return pl.kernel(
        functools.partial(_cascade_body, relu),
        out_type=jax.ShapeDtypeStruct((H, N), f32),
        mesh=mesh,
        scratch_types=(
            [pltpu.VMEM((N,), f32)] * 12
            + [pltpu.VMEM((CH,), i32), pltpu.VMEM((CH,), i32),
               pltpu.VMEM((CH,), f32), pltpu.VMEM((4 * L,), f32)]
        ),
    )


def _mm_xw_t_body(w_ref, x_ref, o_ref):
    o_ref[...] = lax.dot_general(
        w_ref[...], x_ref[...], (((0,), (1,)), ((), ())),
        preferred_element_type=jnp.float32)


def _mm_wt_ht_body(w_ref, h_ref, o_ref):
    o_ref[...] = lax.dot_general(
        w_ref[...], h_ref[...], (((0,), (0,)), ((), ())),
        preferred_element_type=jnp.float32)


def _mm_ht_w_body(h_ref, w_ref, o_ref):
    o_ref[...] = lax.dot_general(
        h_ref[...], w_ref[...], (((0,), (0,)), ((), ())),
        preferred_element_type=jnp.float32)


_NB = 2000


def _tc_mm1(x, W):
    N, D = x.shape
    H = W.shape[1]
    return pl.pallas_call(
        _mm_xw_t_body,
        grid=(N // _NB,),
        in_specs=[pl.BlockSpec((D, H), lambda i: (0, 0)),
                  pl.BlockSpec((_NB, D), lambda i: (i, 0))],
        out_specs=pl.BlockSpec((H, _NB), lambda i: (0, i)),
        out_shape=jax.ShapeDtypeStruct((H, N), jnp.float32),
    )(W, x)


def _tc_mm2(hT, W):
    D, N = hT.shape
    H = W.shape[1]
    return pl.pallas_call(
        _mm_wt_ht_body,
        grid=(N // _NB,),
        in_specs=[pl.BlockSpec((D, H), lambda i: (0, 0)),
                  pl.BlockSpec((D, _NB), lambda i: (0, i))],
        out_specs=pl.BlockSpec((H, _NB), lambda i: (0, i)),
        out_shape=jax.ShapeDtypeStruct((H, N), jnp.float32),
    )(W, hT)


def _tc_mm3(hT, W):
    D, N = hT.shape
    C = W.shape[1]
    return pl.pallas_call(
        _mm_ht_w_body,
        grid=(N // _NB,),
        in_specs=[pl.BlockSpec((D, _NB), lambda i: (0, i)),
                  pl.BlockSpec((D, C), lambda i: (0, 0))],
        out_specs=pl.BlockSpec((_NB, C), lambda i: (i, 0)),
        out_shape=jax.ShapeDtypeStruct((N, C), jnp.float32),
    )(hT, W)


def kernel(x, edge_index, edge_weight, W1, theta1, W2, theta2, W_lin):
    N, D = x.shape
    E = edge_weight.shape[0]
    H = W1.shape[1]
    assert N % L == 0 and E % CH == 0 and (E // NWORK) % CH2 == 0
    assert H == NWORK * FPW and N % _NB == 0

    src = edge_index[0]
    dst = edge_index[1]

    norm = _make_degnorm(E, N)(src, dst, edge_weight)

    h0T = _tc_mm1(x, W1)
    th1 = jnp.repeat(theta1.astype(jnp.float32), L)
    c1T = _make_cascade(E, N, H, True)(h0T, src, dst, norm, th1)

    h0T2 = _tc_mm2(c1T, W2)
    th2 = jnp.repeat(theta2.astype(jnp.float32), L)
    c2T = _make_cascade(E, N, H, False)(h0T2, src, dst, norm, th2)

    return _tc_mm3(c2T, W_lin)


# speedup vs baseline: 9.1704x; 9.1704x over previous
import functools

import jax
import jax.numpy as jnp
from jax import lax
from jax.experimental import pallas as pl
from jax.experimental.pallas import tpu as pltpu
from jax.experimental.pallas import tpu_sc as plsc

L = 16
NWORK = 32
FPW = 4
CH = 1280
CH2 = 2000

_ALPHA = 1.0
_KORD = 3


def _wid():
    c = lax.axis_index("c")
    s = lax.axis_index("s")
    return s * 2 + c


def _rsqrt16(v):
    iv = lax.bitcast_convert_type(v, jnp.int32)
    y = lax.bitcast_convert_type(jnp.int32(0x5F3759DF) - (iv >> 1), jnp.float32)
    for _ in range(3):
        y = y * (1.5 - 0.5 * v * y * y)
    return y


def _stream_pairs(hbm_refs, nch, bufs_a, bufs_b, sem_a, sem_b, compute):
    chn = bufs_a[0].shape[0]

    def fire(c, bufs, sem):
        off = c * chn
        for r, b in zip(hbm_refs, bufs):
            pltpu.async_copy(r.at[pl.ds(off, chn)], b, sem)

    def drain(bufs, sem):
        for r, b in zip(hbm_refs, bufs):
            pltpu.make_async_copy(r.at[pl.ds(0, chn)], b, sem).wait()

    fire(0, bufs_a, sem_a)

    def pair(p, _):
        c0 = 2 * p
        drain(bufs_a, sem_a)
        fire(c0 + 1, bufs_b, sem_b)
        compute(bufs_a)
        drain(bufs_b, sem_b)
        fire(jnp.where(c0 + 2 < nch, c0 + 2, 0), bufs_a, sem_a)
        compute(bufs_b)
        return _
    lax.fori_loop(0, nch // 2, pair, None)
    drain(bufs_a, sem_a)


def _degnorm_body(src_hbm, dst_hbm, ew_hbm, norm_hbm,
                  do_v, di_v, ro_v, ri_v,
                  ea0, eb0, ec0, ea1, eb1, ec1,
                  se2, de2, we2, nb2, sem_a, sem_b):
    E = src_hbm.shape[0]
    N = do_v.shape[0]
    wid = _wid()

    zeros = jnp.zeros((L,), jnp.float32)

    @plsc.parallel_loop(0, N // L, unroll=5)
    def _z(i):
        do_v[pl.ds(i * L, L)] = zeros
        di_v[pl.ds(i * L, L)] = zeros

    def deg_compute(bufs):
        se_, de_, we_ = bufs

        @plsc.parallel_loop(0, CH // L, unroll=4)
        def _g(g):
            sl = pl.ds(g * L, L)
            w16 = we_[sl]
            plsc.addupdate_scatter(do_v, [se_[sl]], w16)
            plsc.addupdate_scatter(di_v, [de_[sl]], w16)

    _stream_pairs((src_hbm, dst_hbm, ew_hbm), E // CH,
                  (ea0, eb0, ec0), (ea1, eb1, ec1), sem_a, sem_b, deg_compute)

    @plsc.parallel_loop(0, N // L, unroll=5)
    def _r(i):
        sl = pl.ds(i * L, L)
        ro_v[sl] = _rsqrt16(do_v[sl] + 1e-6)
        ri_v[sl] = _rsqrt16(di_v[sl] + 1e-6)

    ew_per = E // NWORK

    def n_body(ci, _):
        off = wid * ew_per + ci * CH2
        pltpu.sync_copy(src_hbm.at[pl.ds(off, CH2)], se2)
        pltpu.sync_copy(dst_hbm.at[pl.ds(off, CH2)], de2)
        pltpu.sync_copy(ew_hbm.at[pl.ds(off, CH2)], we2)

        @plsc.parallel_loop(0, CH2 // L, unroll=5)
        def _g(g):
            sl = pl.ds(g * L, L)
            nb2[sl] = (we2[sl]
                       * plsc.load_gather(ro_v, [se2[sl]])
                       * plsc.load_gather(ri_v, [de2[sl]]))
        pltpu.sync_copy(nb2, norm_hbm.at[pl.ds(off, CH2)])
        return _
    lax.fori_loop(0, ew_per // CH2, n_body, None)


@functools.lru_cache(maxsize=None)
def _make_degnorm(E, N):
    mesh = plsc.VectorSubcoreMesh(core_axis_name="c", subcore_axis_name="s")
    f32, i32 = jnp.float32, jnp.int32
    return pl.kernel(
        _degnorm_body,
        out_type=jax.ShapeDtypeStruct((E,), f32),
        mesh=mesh,
        compiler_params=pltpu.CompilerParams(needs_layout_passes=False),
        scratch_types=[
            pltpu.VMEM((N,), f32), pltpu.VMEM((N,), f32),
            pltpu.VMEM((N,), f32), pltpu.VMEM((N,), f32),
            pltpu.VMEM((CH,), i32), pltpu.VMEM((CH,), i32), pltpu.VMEM((CH,), f32),
            pltpu.VMEM((CH,), i32), pltpu.VMEM((CH,), i32), pltpu.VMEM((CH,), f32),
            pltpu.VMEM((CH2,), i32), pltpu.VMEM((CH2,), i32), pltpu.VMEM((CH2,), f32),
            pltpu.VMEM((CH2,), f32),
            pltpu.SemaphoreType.DMA, pltpu.SemaphoreType.DMA,
        ],
    )


def _cascade_body(relu, h0_hbm, src_hbm, dst_hbm, norm_hbm, th_hbm, out_hbm,
                  t0, t1, t2, t3, u0, u1, u2, u3, c0, c1, c2, c3,
                  ea0, eb0, ec0, ea1, eb1, ec1, th, sem_a, sem_b):
    E = src_hbm.shape[0]
    N = t0.shape[0]
    wid = _wid()
    ts = (t0, t1, t2, t3)
    us = (u0, u1, u2, u3)
    cs = (c0, c1, c2, c3)

    for j in range(FPW):
        pltpu.sync_copy(h0_hbm.at[wid * FPW + j], ts[j])
    pltpu.sync_copy(th_hbm, th)

    zeros = jnp.zeros((L,), jnp.float32)

    @plsc.parallel_loop(0, N // L, unroll=5)
    def _z(i):
        for j in range(FPW):
            us[j][pl.ds(i * L, L)] = zeros

    def prop(tables, accs, coef):
        def compute(bufs):
            se_, de_, ne_ = bufs

            @plsc.parallel_loop(0, CH // L, unroll=4)
            def _g(g):
                sl = pl.ds(g * L, L)
                s16 = se_[sl]
                d16 = de_[sl]
                n16 = ne_[sl] * coef
                for j in range(FPW):
                    v = plsc.load_gather(tables[j], [s16]) * n16
                    plsc.addupdate_scatter(accs[j], [d16], v)

        _stream_pairs((src_hbm, dst_hbm, norm_hbm), E // CH,
                      (ea0, eb0, ec0), (ea1, eb1, ec1), sem_a, sem_b, compute)

    th_v = [th[pl.ds(k * L, L)] for k in range(_KORD + 1)]

    prop(ts, us, 2.0 * _ALPHA)

    @plsc.parallel_loop(0, N // L, unroll=5)
    def _ci(i):
        sl = pl.ds(i * L, L)
        for j in range(FPW):
            cs[j][sl] = th_v[0] * ts[j][sl] + th_v[1] * us[j][sl]

    for k in range(2, _KORD + 1):
        a_k = 2.0 * (k - 1.0 + _ALPHA) / k
        b_k = (k - 2.0 + 2.0 * _ALPHA) / k
        tables, accs = (us, ts) if k % 2 == 0 else (ts, us)

        @plsc.parallel_loop(0, N // L, unroll=5)
        def _pre(i, accs=accs, b_k=b_k):
            sl = pl.ds(i * L, L)
            for j in range(FPW):
                accs[j][sl] = accs[j][sl] * (-b_k)

        prop(tables, accs, a_k)

        if k == _KORD and relu:
            @plsc.parallel_loop(0, N // L, unroll=5)
            def _cl(i, accs=accs, k=k):
                sl = pl.ds(i * L, L)
                for j in range(FPW):
                    cs[j][sl] = jnp.maximum(
                        cs[j][sl] + th_v[k] * accs[j][sl], 0.0)
        else:
            @plsc.parallel_loop(0, N // L, unroll=5)
            def _ck(i, accs=accs, k=k):
                sl = pl.ds(i * L, L)
                for j in range(FPW):
                    cs[j][sl] = cs[j][sl] + th_v[k] * accs[j][sl]

    for j in range(FPW):
        pltpu.sync_copy(cs[j], out_hbm.at[wid * FPW + j])


@functools.lru_cache(maxsize=None)
def _make_cascade(E, N, H, relu):
    mesh = plsc.VectorSubcoreMesh(core_axis_name="c", subcore_axis_name="s")
    f32, i32 = jnp.float32, jnp.int32
    return pl.kernel(
        functools.partial(_cascade_body, relu),
        out_type=jax.ShapeDtypeStruct((H, N), f32),
        mesh=mesh,
        compiler_params=pltpu.CompilerParams(needs_layout_passes=False),
        scratch_types=(
            [pltpu.VMEM((N,), f32)] * 12
            + [pltpu.VMEM((CH,), i32), pltpu.VMEM((CH,), i32),
               pltpu.VMEM((CH,), f32),
               pltpu.VMEM((CH,), i32), pltpu.VMEM((CH,), i32),
               pltpu.VMEM((CH,), f32),
               pltpu.VMEM((4 * L,), f32),
               pltpu.SemaphoreType.DMA, pltpu.SemaphoreType.DMA]
        ),
    )


def _mm_xw_t_body(w_ref, x_ref, o_ref):
    o_ref[...] = lax.dot_general(
        w_ref[...], x_ref[...], (((0,), (1,)), ((), ())),
        preferred_element_type=jnp.float32)


def _mm_wt_ht_body(w_ref, h_ref, o_ref):
    o_ref[...] = lax.dot_general(
        w_ref[...], h_ref[...], (((0,), (0,)), ((), ())),
        preferred_element_type=jnp.float32)


def _mm_ht_w_body(h_ref, w_ref, o_ref):
    o_ref[...] = lax.dot_general(
        h_ref[...], w_ref[...], (((0,), (0,)), ((), ())),
        preferred_element_type=jnp.float32)


_NB = 2048


def _tc_mm1(x, W):
    N, D = x.shape
    H = W.shape[1]
    return pl.pallas_call(
        _mm_xw_t_body,
        grid=(pl.cdiv(N, _NB),),
        in_specs=[pl.BlockSpec((D, H), lambda i: (0, 0)),
                  pl.BlockSpec((_NB, D), lambda i: (i, 0))],
        out_specs=pl.BlockSpec((H, _NB), lambda i: (0, i)),
        out_shape=jax.ShapeDtypeStruct((H, N), jnp.float32),
    )(W, x)


def _tc_mm2(hT, W):
    D, N = hT.shape
    H = W.shape[1]
    return pl.pallas_call(
        _mm_wt_ht_body,
        grid=(pl.cdiv(N, _NB),),
        in_specs=[pl.BlockSpec((D, H), lambda i: (0, 0)),
                  pl.BlockSpec((D, _NB), lambda i: (0, i))],
        out_specs=pl.BlockSpec((H, _NB), lambda i: (0, i)),
        out_shape=jax.ShapeDtypeStruct((H, N), jnp.float32),
    )(W, hT)


def _tc_mm3(hT, W):
    D, N = hT.shape
    C = W.shape[1]
    return pl.pallas_call(
        _mm_ht_w_body,
        grid=(pl.cdiv(N, _NB),),
        in_specs=[pl.BlockSpec((D, _NB), lambda i: (0, i)),
                  pl.BlockSpec((D, C), lambda i: (0, 0))],
        out_specs=pl.BlockSpec((_NB, C), lambda i: (i, 0)),
        out_shape=jax.ShapeDtypeStruct((N, C), jnp.float32),
    )(hT, W)


def kernel(x, edge_index, edge_weight, W1, theta1, W2, theta2, W_lin):
    N, D = x.shape
    E = edge_weight.shape[0]
    H = W1.shape[1]
    assert N % L == 0 and E % (2 * CH) == 0 and (E // NWORK) % CH2 == 0
    assert H == NWORK * FPW

    src = edge_index[0]
    dst = edge_index[1]

    norm = _make_degnorm(E, N)(src, dst, edge_weight)

    h0T = _tc_mm1(x, W1)
    th1 = jnp.repeat(theta1.astype(jnp.float32), L)
    c1T = _make_cascade(E, N, H, True)(h0T, src, dst, norm, th1)

    h0T2 = _tc_mm2(c1T, W2)
    th2 = jnp.repeat(theta2.astype(jnp.float32), L)
    c2T = _make_cascade(E, N, H, False)(h0T2, src, dst, norm, th2)

    return _tc_mm3(c2T, W_lin)
